# baseline (speedup 1.0000x reference)
import jax
import jax.numpy as jnp
from jax.experimental import pallas as pl
from jax.experimental.pallas import tpu as pltpu


_TM = 512
_NBUF = 6


def _interp_pipeline(x_ref, s_hbm, o_ref, buf, sem):
    nsteps = s_hbm.shape[0] // _TM

    def copy_in(step):
        slot = step % _NBUF
        return pltpu.make_async_copy(
            s_hbm.at[pl.ds(step * _TM, _TM), :],
            buf.at[slot],
            sem.at[slot],
        )

    for j in range(_NBUF):
        copy_in(j).start()

    for i in range(nsteps):
        copy_in(i).wait()
        o_ref[pl.ds(i * _TM, _TM), :] = jnp.dot(
            buf[i % _NBUF], x_ref[...], preferred_element_type=jnp.float32)
        if i + _NBUF < nsteps:
            copy_in(i + _NBUF).start()


def kernel(x_coarse, interp_matrix):
    m, k = interp_matrix.shape
    n = x_coarse.shape[1]
    return pl.pallas_call(
        _interp_pipeline,
        in_specs=[
            pl.BlockSpec(memory_space=pltpu.MemorySpace.VMEM),
            pl.BlockSpec(memory_space=pl.ANY),
        ],
        out_specs=pl.BlockSpec(memory_space=pltpu.MemorySpace.VMEM),
        out_shape=jax.ShapeDtypeStruct((m, n), jnp.float32),
        scratch_shapes=[
            pltpu.VMEM((_NBUF, _TM, 4096), jnp.float32),
            pltpu.SemaphoreType.DMA((_NBUF,)),
        ],
    )(x_coarse, interp_matrix)
